# Initial kernel scaffold; baseline (speedup 1.0000x reference)
import jax
import jax.numpy as jnp
from jax.experimental import pallas as pl


def kernel(input, p_mat, weight):
    raise NotImplementedError("write your pallas kernel here")


# speedup vs baseline: 1.0418x; 1.0418x over previous
import functools

import jax
import jax.numpy as jnp
from jax.experimental import pallas as pl
from jax.experimental.pallas import tpu as pltpu

N = 10000
BM = 400


def _gcn_kernel(x_ref, w_ref, p_ref, o_ref, s_ref):
    @pl.when(pl.program_id(0) == 0)
    def _():
        s_ref[...] = jnp.dot(
            x_ref[...], w_ref[...], preferred_element_type=jnp.float32
        )

    o_ref[...] = jnp.dot(
        p_ref[...], s_ref[...], preferred_element_type=jnp.float32
    )


@jax.jit
def kernel(input, p_mat, weight):
    n, d_in = input.shape
    d_out = weight.shape[1]
    grid = (n // BM,)
    return pl.pallas_call(
        _gcn_kernel,
        grid=grid,
        in_specs=[
            pl.BlockSpec((n, d_in), lambda i: (0, 0)),
            pl.BlockSpec((d_in, d_out), lambda i: (0, 0)),
            pl.BlockSpec((BM, n), lambda i: (i, 0)),
        ],
        out_specs=pl.BlockSpec((BM, d_out), lambda i: (i, 0)),
        out_shape=jax.ShapeDtypeStruct((n, d_out), jnp.float32),
        scratch_shapes=[pltpu.VMEM((n, d_out), jnp.float32)],
    )(input, weight, p_mat)
